# Initial kernel scaffold; baseline (speedup 1.0000x reference)
import functools

import jax
import jax.numpy as jnp
from jax import lax
from jax.experimental import pallas as pl
from jax.experimental.pallas import tpu as pltpu
from jax.experimental.pallas import tpu_sc as plsc

N = 10000
E = 320000
D = 128

NC = 2
NS = 16
NW = NC * NS

K = 128
S = (E + NW * K - 1) // (NW * K)
E_PAD = NW * S * K
NROWS = 10048
PAD_ROW = 10016
RPT = NROWS // NS
HCHUNK = E // 8
QCOL = NROWS // 4

_mesh = plsc.VectorSubcoreMesh(core_axis_name="c", subcore_axis_name="s")


@functools.partial(
    pl.kernel,
    out_type=jax.ShapeDtypeStruct((NC, 4, NROWS), jnp.float32),
    mesh=_mesh,
    scratch_types=[
        pltpu.VMEM((HCHUNK,), jnp.int32),
        pltpu.VMEM((NROWS,), jnp.float32),
        pltpu.VMEM((4, QCOL), jnp.float32),
        pltpu.VMEM((QCOL,), jnp.float32),
        pltpu.VMEM_SHARED((NS, NROWS), jnp.float32),
    ],
)
def _hist_sc(idx4, hist_out, idx_v, hist_v, rbuf, obuf, shared_h):
    c = lax.axis_index("c")
    s = lax.axis_index("s")
    a = s // 4
    part = (s % 4) * NC + c

    @pl.loop(0, NROWS // 16)
    def _zero(i):
        hist_v[pl.ds(i * 16, 16)] = jnp.zeros((16,), jnp.float32)

    pltpu.sync_copy(idx4.at[a, pl.ds(part * HCHUNK, HCHUNK)], idx_v)
    ones = jnp.ones((16,), jnp.float32)

    @pl.loop(0, HCHUNK // 16)
    def _count(j):
        iv = idx_v[pl.ds(j * 16, 16)]
        plsc.addupdate_scatter(hist_v, [iv], ones)

    pltpu.sync_copy(hist_v, shared_h.at[s])
    plsc.subcore_barrier()

    base = (s % 4) * QCOL
    for l in range(4):
        pltpu.sync_copy(shared_h.at[a * 4 + l, pl.ds(base, QCOL)], rbuf.at[l])

    @pl.loop(0, QCOL // 16)
    def _red(i):
        sl = pl.ds(i * 16, 16)
        obuf[sl] = rbuf[0, sl] + rbuf[1, sl] + rbuf[2, sl] + rbuf[3, sl]

    pltpu.sync_copy(obuf, hist_out.at[c, a, pl.ds(base, QCOL)])


@functools.partial(
    pl.kernel,
    out_type=jax.ShapeDtypeStruct((NC, NROWS, D), jnp.float32),
    mesh=_mesh,
    scratch_types=[
        pltpu.VMEM((S, K), jnp.int32),
        pltpu.VMEM((S, K), jnp.int32),
        pltpu.VMEM((K, D), jnp.float32),
        pltpu.VMEM_SHARED((NROWS, D), jnp.float32),
        pltpu.SemaphoreType.DMA,
    ],
)
def _agg_sc(h_hbm, src_b, dst_b, zeros_hbm, part_out,
            src_v, dst_v, rows_v, accum, gsem):
    c = lax.axis_index("c")
    s = lax.axis_index("s")
    g = s * NC + c

    pltpu.sync_copy(zeros_hbm.at[pl.ds(s * RPT, RPT)], accum.at[pl.ds(s * RPT, RPT)])
    pltpu.sync_copy(src_b.at[g], src_v)
    pltpu.sync_copy(dst_b.at[g], dst_v)
    plsc.subcore_barrier()

    @pl.loop(0, S)
    def _step(j):
        pltpu.async_copy(h_hbm.at[src_v.at[j]], rows_v, gsem).wait()
        pltpu.sync_copy(rows_v, accum.at[dst_v.at[j]], add=True)

    plsc.subcore_barrier()
    pltpu.sync_copy(accum.at[pl.ds(s * RPT, RPT)],
                    part_out.at[c, pl.ds(s * RPT, RPT)])


def _mm1_body(x_ref, w_ref, hist_ref, h_ref):
    cnt = hist_ref[0, 0, :N] + hist_ref[1, 0, :N]
    scale = lax.rsqrt(jnp.maximum(cnt, 1.0))
    h_ref[...] = jnp.dot(x_ref[...] * scale[:, None], w_ref[...],
                         preferred_element_type=jnp.float32)


def _bn(h, gamma, beta):
    mu = jnp.mean(h, axis=0)
    var = jnp.mean((h - mu[None, :]) ** 2, axis=0)
    return gamma[None, :] * (h - mu[None, :]) * lax.rsqrt(var + 1e-5) + beta[None, :]


def _fused2_body(p_ref, hist_ref, b1_ref, g1_ref, be1_ref, mask_ref, w2_ref, h2_ref):
    agg = p_ref[0, :N, :] + p_ref[1, :N, :]
    cin = hist_ref[0, 1, :N] + hist_ref[1, 1, :N]
    agg = agg * lax.rsqrt(jnp.maximum(cin, 1.0))[:, None] + b1_ref[...][None, :]
    y = jnp.maximum(_bn(agg, g1_ref[...], be1_ref[...]), 0.0)
    y = y * mask_ref[...]
    cout2 = hist_ref[0, 2, :N] + hist_ref[1, 2, :N]
    y = y * lax.rsqrt(jnp.maximum(cout2, 1.0))[:, None]
    h2_ref[...] = jnp.dot(y, w2_ref[...], preferred_element_type=jnp.float32)


def _fused3_body(p_ref, hist_ref, b2_ref, g2_ref, be2_ref, out_ref):
    agg = p_ref[0, :N, :] + p_ref[1, :N, :]
    cin = hist_ref[0, 3, :N] + hist_ref[1, 3, :N]
    agg = agg * lax.rsqrt(jnp.maximum(cin, 1.0))[:, None] + b2_ref[...][None, :]
    out_ref[...] = jnp.maximum(_bn(agg, g2_ref[...], be2_ref[...]), 0.0)


_mm1 = pl.pallas_call(_mm1_body, out_shape=jax.ShapeDtypeStruct((N, D), jnp.float32))
_fused2 = pl.pallas_call(_fused2_body, out_shape=jax.ShapeDtypeStruct((N, D), jnp.float32))
_fused3 = pl.pallas_call(_fused3_body, out_shape=jax.ShapeDtypeStruct((N, D), jnp.float32))


def _prep_edges(idx, fill):
    pad = jnp.full((E_PAD - E,), fill, jnp.int32)
    return jnp.reshape(jnp.concatenate([idx, pad]), (NW, S, K))


def kernel(x, edge_index1, edge_index2, W1, b1, W2, b2,
           bn1_gamma, bn1_beta, bn2_gamma, bn2_beta):
    idx4 = jnp.concatenate([edge_index1, edge_index2], axis=0)
    hist = _hist_sc(idx4)

    h1 = _mm1(x, W1, hist)

    zeros = jnp.zeros((NROWS, D), jnp.float32)
    src1 = _prep_edges(edge_index1[0], 0)
    dst1 = _prep_edges(edge_index1[1], PAD_ROW)
    p1 = _agg_sc(h1, src1, dst1, zeros)

    mask = jax.random.bernoulli(jax.random.key(42), 0.5, (N, D))
    maskscale = jnp.where(mask, 2.0, 0.0).astype(jnp.float32)
    h2 = _fused2(p1, hist, b1, bn1_gamma, bn1_beta, maskscale, W2)

    src2 = _prep_edges(edge_index2[0], 0)
    dst2 = _prep_edges(edge_index2[1], PAD_ROW)
    p2 = _agg_sc(h2, src2, dst2, zeros)

    return _fused3(p2, hist, b2, bn2_gamma, bn2_beta)


# speedup vs baseline: 4.3005x; 4.3005x over previous
import functools

import jax
import jax.numpy as jnp
from jax import lax
from jax.experimental import pallas as pl
from jax.experimental.pallas import tpu as pltpu
from jax.experimental.pallas import tpu_sc as plsc

N = 10000
E = 320000
D = 128

NC = 2
NS = 16
NW = NC * NS

K = 128
S = (E + NW * K - 1) // (NW * K)
E_PAD = NW * S * K
NROWS = 10240
PAD_ROW = 10016
RPT = NROWS // NS
QCOL = NROWS // 4
HROWS = 320
H_PAD = HROWS * 128 * 8

_mesh = plsc.VectorSubcoreMesh(core_axis_name="c", subcore_axis_name="s")


@functools.partial(
    pl.kernel,
    out_type=jax.ShapeDtypeStruct((NW, 1, QCOL), jnp.float32),
    mesh=_mesh,
    compiler_params=pltpu.CompilerParams(needs_layout_passes=False),
    scratch_types=[
        pltpu.VMEM((HROWS, 128), jnp.int32),
        pltpu.VMEM((NROWS,), jnp.float32),
        pltpu.VMEM((4, QCOL), jnp.float32),
        pltpu.VMEM((QCOL,), jnp.float32),
        pltpu.VMEM_SHARED((NS, 1, NROWS), jnp.float32),
    ],
)
def _hist_sc(idx4, hist_out, idx_v, hist_v, rbuf, obuf, shared_h):
    c = lax.axis_index("c")
    s = lax.axis_index("s")
    a = s // 4
    q = s % 4
    part = a * 8 + q * NC + c

    @pl.loop(0, NROWS // 16)
    def _zero(i):
        hist_v[pl.ds(i * 16, 16)] = jnp.zeros((16,), jnp.float32)

    pltpu.sync_copy(idx4.at[part], idx_v)
    ones = jnp.ones((16,), jnp.float32)

    @pl.loop(0, HROWS * 8)
    def _count(t):
        iv = idx_v[t // 8, pl.ds((t % 8) * 16, 16)]
        plsc.addupdate_scatter(hist_v, [iv], ones)

    pltpu.sync_copy(hist_v, shared_h.at[s, 0])
    plsc.subcore_barrier()

    base = q * QCOL
    for l in range(4):
        pltpu.sync_copy(shared_h.at[a * 4 + l, 0, pl.ds(base, QCOL)], rbuf.at[l])

    @pl.loop(0, QCOL // 16)
    def _red(i):
        sl = pl.ds(i * 16, 16)
        obuf[sl] = rbuf[0, sl] + rbuf[1, sl] + rbuf[2, sl] + rbuf[3, sl]

    pltpu.sync_copy(obuf, hist_out.at[c * 16 + a * 4 + q, 0])


@functools.partial(
    pl.kernel,
    out_type=jax.ShapeDtypeStruct((NC, NROWS, D), jnp.float32),
    mesh=_mesh,
    scratch_types=[
        pltpu.VMEM((S, K), jnp.int32),
        pltpu.VMEM((S, K), jnp.int32),
        pltpu.VMEM((K, D), jnp.float32),
        pltpu.VMEM_SHARED((NROWS, D), jnp.float32),
        pltpu.SemaphoreType.DMA,
    ],
)
def _agg_sc(h_hbm, src_b, dst_b, zeros_hbm, part_out,
            src_v, dst_v, rows_v, accum, gsem):
    c = lax.axis_index("c")
    s = lax.axis_index("s")
    g = s * NC + c

    pltpu.sync_copy(zeros_hbm.at[pl.ds(s * RPT, RPT)], accum.at[pl.ds(s * RPT, RPT)])
    pltpu.sync_copy(src_b.at[g], src_v)
    pltpu.sync_copy(dst_b.at[g], dst_v)
    plsc.subcore_barrier()

    @pl.loop(0, S)
    def _step(j):
        pltpu.async_copy(h_hbm.at[src_v.at[j]], rows_v, gsem).wait()
        pltpu.sync_copy(rows_v, accum.at[dst_v.at[j]], add=True)

    plsc.subcore_barrier()
    pltpu.sync_copy(accum.at[pl.ds(s * RPT, RPT)],
                    part_out.at[c, pl.ds(s * RPT, RPT)])


def _counts(hist_arr, a):
    h = hist_arr.reshape(2, 4, 4 * QCOL)
    return (h[0, a] + h[1, a])[:N]


def _mm1_body(x_ref, w_ref, hist_ref, h_ref):
    scale = lax.rsqrt(jnp.maximum(_counts(hist_ref[...], 0), 1.0))
    h_ref[...] = jnp.dot(x_ref[...] * scale[:, None], w_ref[...],
                         preferred_element_type=jnp.float32)


def _bn(h, gamma, beta):
    mu = jnp.mean(h, axis=0)
    var = jnp.mean((h - mu[None, :]) ** 2, axis=0)
    return gamma[None, :] * (h - mu[None, :]) * lax.rsqrt(var + 1e-5) + beta[None, :]


def _fused2_body(p_ref, hist_ref, b1_ref, g1_ref, be1_ref, mask_ref, w2_ref, h2_ref):
    agg = p_ref[0, :N, :] + p_ref[1, :N, :]
    cin = _counts(hist_ref[...], 1)
    agg = agg * lax.rsqrt(jnp.maximum(cin, 1.0))[:, None] + b1_ref[...][None, :]
    y = jnp.maximum(_bn(agg, g1_ref[...], be1_ref[...]), 0.0)
    y = y * mask_ref[...]
    cout2 = _counts(hist_ref[...], 2)
    y = y * lax.rsqrt(jnp.maximum(cout2, 1.0))[:, None]
    h2_ref[...] = jnp.dot(y, w2_ref[...], preferred_element_type=jnp.float32)


def _fused3_body(p_ref, hist_ref, b2_ref, g2_ref, be2_ref, out_ref):
    agg = p_ref[0, :N, :] + p_ref[1, :N, :]
    cin = _counts(hist_ref[...], 3)
    agg = agg * lax.rsqrt(jnp.maximum(cin, 1.0))[:, None] + b2_ref[...][None, :]
    out_ref[...] = jnp.maximum(_bn(agg, g2_ref[...], be2_ref[...]), 0.0)


_mm1 = pl.pallas_call(_mm1_body, out_shape=jax.ShapeDtypeStruct((N, D), jnp.float32))
_fused2 = pl.pallas_call(_fused2_body, out_shape=jax.ShapeDtypeStruct((N, D), jnp.float32))
_fused3 = pl.pallas_call(_fused3_body, out_shape=jax.ShapeDtypeStruct((N, D), jnp.float32))


def _prep_edges(idx, fill):
    pad = jnp.full((E_PAD - E,), fill, jnp.int32)
    return jnp.reshape(jnp.concatenate([idx, pad]), (NW, S, K))


def kernel(x, edge_index1, edge_index2, W1, b1, W2, b2,
           bn1_gamma, bn1_beta, bn2_gamma, bn2_beta):
    hpad = jnp.full((4, H_PAD - E), PAD_ROW, jnp.int32)
    idx4 = jnp.concatenate(
        [jnp.concatenate([edge_index1, edge_index2], axis=0), hpad], axis=1)
    idx4 = idx4.reshape(NW, HROWS, 128)
    hist = _hist_sc(idx4)

    h1 = _mm1(x, W1, hist)

    zeros = jnp.zeros((NROWS, D), jnp.float32)
    src1 = _prep_edges(edge_index1[0], 0)
    dst1 = _prep_edges(edge_index1[1], PAD_ROW)
    p1 = _agg_sc(h1, src1, dst1, zeros)

    mask = jax.random.bernoulli(jax.random.key(42), 0.5, (N, D))
    maskscale = jnp.where(mask, 2.0, 0.0).astype(jnp.float32)
    h2 = _fused2(p1, hist, b1, bn1_gamma, bn1_beta, maskscale, W2)

    src2 = _prep_edges(edge_index2[0], 0)
    dst2 = _prep_edges(edge_index2[1], PAD_ROW)
    p2 = _agg_sc(h2, src2, dst2, zeros)

    return _fused3(p2, hist, b2, bn2_gamma, bn2_beta)
